# Initial kernel scaffold; baseline (speedup 1.0000x reference)
import jax
import jax.numpy as jnp
from jax.experimental import pallas as pl


def kernel(camera_ids, weight):
    raise NotImplementedError("write your pallas kernel here")


# speedup vs baseline: 1.0651x; 1.0651x over previous
import functools

import jax
import jax.numpy as jnp
from jax import lax
from jax.experimental import pallas as pl
from jax.experimental.pallas import tpu as pltpu
from jax.experimental.pallas import tpu_sc as plsc

D = 32
B = 16384 * 50

NC = 2
NS = 16
NW = NC * NS
B_PER_W = B // NW

G = 128
K = 8
CHUNK = K * G
N_STEPS = B_PER_W // CHUNK

_mesh = plsc.VectorSubcoreMesh(core_axis_name="c", subcore_axis_name="s")


@functools.partial(
    pl.kernel,
    mesh=_mesh,
    out_type=jax.ShapeDtypeStruct((B, D), jnp.float32),
    scratch_types=[
        pltpu.VMEM((K, G), jnp.int32),
        pltpu.VMEM((CHUNK, D), jnp.float32),
        pltpu.SemaphoreType.DMA,
    ],
    compiler_params=pltpu.CompilerParams(use_tc_tiling_on_sc=False),
)
def _gather_kernel(idx_hbm, table_hbm, out_hbm, idx_v, rows_v, gsem):
    wid = lax.axis_index("s") * NC + lax.axis_index("c")
    row0 = wid * (B_PER_W // G)
    base = wid * B_PER_W

    def step(i, _):
        pltpu.sync_copy(idx_hbm.at[pl.ds(row0 + i * K, K)], idx_v)
        copies = [
            pltpu.async_copy(
                table_hbm.at[idx_v.at[j]],
                rows_v.at[pl.ds(j * G, G)],
                gsem,
            )
            for j in range(K)
        ]
        for c in copies:
            c.wait()
        pltpu.sync_copy(rows_v, out_hbm.at[pl.ds(base + i * CHUNK, CHUNK)])
        return 0

    lax.fori_loop(0, N_STEPS, step, 0)


def kernel(camera_ids, weight):
    ids = camera_ids.reshape(-1).astype(jnp.int32)
    idx2d = ids.reshape(B // G, G)
    return _gather_kernel(idx2d, weight)


# speedup vs baseline: 1.0944x; 1.0275x over previous
import functools

import jax
import jax.numpy as jnp
from jax import lax
from jax.experimental import pallas as pl
from jax.experimental.pallas import tpu as pltpu
from jax.experimental.pallas import tpu_sc as plsc

D = 32
B = 16384 * 50

NC = 2
NS = 16
NW = NC * NS
B_PER_W = B // NW

G = 128
K = 8
CHUNK = K * G
N_STEPS = B_PER_W // CHUNK

_mesh = plsc.VectorSubcoreMesh(core_axis_name="c", subcore_axis_name="s")


@functools.partial(
    pl.kernel,
    mesh=_mesh,
    out_type=jax.ShapeDtypeStruct((B, D), jnp.float32),
    scratch_types=[
        pltpu.VMEM((K, G), jnp.int32),
        pltpu.VMEM((K, G), jnp.int32),
        pltpu.VMEM((CHUNK, D), jnp.float32),
        pltpu.VMEM((CHUNK, D), jnp.float32),
        pltpu.SemaphoreType.DMA,
        pltpu.SemaphoreType.DMA,
        pltpu.SemaphoreType.DMA,
        pltpu.SemaphoreType.DMA,
        pltpu.SemaphoreType.DMA,
        pltpu.SemaphoreType.DMA,
    ],
    compiler_params=pltpu.CompilerParams(use_tc_tiling_on_sc=False),
)
def _gather_kernel(idx_hbm, table_hbm, out_hbm,
                   idx0, idx1, rows0, rows1,
                   is0, is1, gs0, gs1, os0, os1):
    wid = lax.axis_index("s") * NC + lax.axis_index("c")
    row0 = wid * (B_PER_W // G)
    base = wid * B_PER_W

    idx_v = (idx0, idx1)
    rows_v = (rows0, rows1)
    isem = (is0, is1)
    gsem = (gs0, gs1)
    osem = (os0, os1)

    def idx_copy(g, b):
        return pltpu.make_async_copy(
            idx_hbm.at[pl.ds(row0 + g * K, K)], idx_v[b], isem[b])

    def gather_copy(j, b):
        return pltpu.make_async_copy(
            table_hbm.at[idx_v[b].at[j]],
            rows_v[b].at[pl.ds(j * G, G)], gsem[b])

    def out_copy(g, b):
        return pltpu.make_async_copy(
            rows_v[b], out_hbm.at[pl.ds(base + g * CHUNK, CHUNK)], osem[b])

    def fire_gathers(b):
        for j in range(K):
            gather_copy(j, b).start()

    def drain_gathers(b):
        for j in range(K):
            gather_copy(j, b).wait()

    idx_copy(0, 0).start()
    idx_copy(1, 1).start()
    idx_copy(0, 0).wait()
    fire_gathers(0)

    def pair(t, _):
        h1 = 2 * t + 1
        idx_copy(h1, 1).wait()

        @pl.when(t > 0)
        def _():
            out_copy(h1 - 2, 1).wait()

        fire_gathers(1)
        drain_gathers(0)
        out_copy(h1 - 1, 0).start()
        idx_copy(h1 + 1, 0).start()

        h2 = 2 * t + 2
        idx_copy(h2, 0).wait()
        out_copy(h2 - 2, 0).wait()
        fire_gathers(0)
        drain_gathers(1)
        out_copy(h2 - 1, 1).start()

        @pl.when(h2 + 1 < N_STEPS)
        def _():
            idx_copy(h2 + 1, 1).start()

        return 0

    lax.fori_loop(0, (N_STEPS - 1) // 2, pair, 0)

    drain_gathers(0)
    out_copy(N_STEPS - 1, 0).start()
    out_copy(N_STEPS - 2, 1).wait()
    out_copy(N_STEPS - 1, 0).wait()


def kernel(camera_ids, weight):
    ids = camera_ids.reshape(-1).astype(jnp.int32)
    idx2d = ids.reshape(B // G, G)
    return _gather_kernel(idx2d, weight)
